# baseline (speedup 1.0000x reference)
import functools

import numpy as np
import jax
import jax.numpy as jnp
from jax import lax
from jax.experimental import pallas as pl
from jax.experimental.pallas import tpu as pltpu
from jax.experimental.pallas import tpu_sc as plsc

_S, _B, _D, _DR = 577, 64, 768, 4
_DRP = 16
_NT = _S - 1
_K = _NT // 2
_KO = _K + 1
_ROWS = _KO * _B
_NC, _NS = 2, 16
_NW = _NC * _NS
_PW_LO, _PW_HI = 576, 584
_NW_LO = 24
_CHUNK = 64
_NBUF = 2
_TAIL = _PW_HI - _PW_LO
_L = 16
_KPAD = 304
_BW = _B // _NW
_SEG = _DR * _S

def _keep_row_table() -> np.ndarray:
    def threefry2x32(k1, k2, x0, x1):
        def rotl(x, r):
            return ((x << np.uint32(r)) | (x >> np.uint32(32 - r))).astype(np.uint32)
        ks = [np.uint32(k1), np.uint32(k2), np.uint32(k1 ^ k2 ^ 0x1BD11BDA)]
        rotations = ((13, 15, 26, 6), (17, 29, 16, 24))
        x0 = (x0 + ks[0]).astype(np.uint32)
        x1 = (x1 + ks[1]).astype(np.uint32)
        for i in range(5):
            for r in rotations[i % 2]:
                x0 = (x0 + x1).astype(np.uint32)
                x1 = rotl(x1, r)
                x1 = (x1 ^ x0).astype(np.uint32)
            x0 = (x0 + ks[(i + 1) % 3]).astype(np.uint32)
            x1 = (x1 + ks[(i + 2) % 3] + np.uint32(i + 1)).astype(np.uint32)
        return x0, x1

    idx = np.arange(_B * _NT, dtype=np.uint64)
    o0, o1 = threefry2x32(0, 1, (idx >> np.uint64(32)).astype(np.uint32),
                          (idx & np.uint64(0xFFFFFFFF)).astype(np.uint32))
    keys = ((o0 ^ o1) >> np.uint32(9)).astype(np.int64).reshape(_B, _NT)
    keep = np.argsort(-keys, axis=1, kind="stable")[:, :_K]
    cols = np.arange(_B, dtype=np.int32)[None, :]
    kept = keep.T.astype(np.int32) * _B + cols
    cls = np.broadcast_to((_S - 1) * _B + cols, (_KPAD - _K, _B))
    return np.concatenate([kept, cls], axis=0).reshape(-1).astype(np.int32)


_KEEP_ROWS = _keep_row_table()


def _worker_base():
    wid = lax.axis_index("s") * _NC + lax.axis_index("c")
    base = wid * _PW_LO + jnp.maximum(wid - _NW_LO, 0) * _TAIL
    return wid, base


def _sc_gather(xf, roif, g):
    mesh = plsc.VectorSubcoreMesh(core_axis_name="c", subcore_axis_name="s")
    nfull = _PW_LO // _CHUNK

    @functools.partial(
        pl.kernel,
        mesh=mesh,
        out_type=(
            jax.ShapeDtypeStruct((_ROWS, _D), jnp.float32),
            jax.ShapeDtypeStruct((_B * _DR * _KO,), jnp.float32),
        ),
        scratch_types=[
            pltpu.VMEM((_KPAD * _B,), jnp.int32),
            pltpu.VMEM((_NBUF, _CHUNK, _D), jnp.float32),
            pltpu.VMEM((_BW * _SEG,), jnp.float32),
            pltpu.VMEM((_BW * _DR * _KO,), jnp.float32),
            pltpu.SemaphoreType.DMA,
            [pltpu.SemaphoreType.DMA] * _NBUF,
        ],
        compiler_params=pltpu.CompilerParams(needs_layout_passes=False),
    )
    def k(x_hbm, roif_hbm, g_hbm, outx_hbm, outroi_hbm,
          g_v, buf_v, in_v, out_v, semx, semo):
        wid, base = _worker_base()
        pltpu.sync_copy(g_hbm, g_v)

        sizes = [16, 32] + [_CHUNK] * ((_PW_LO - 16 - 32 - 16) // _CHUNK) + [16]
        assert sum(sizes) == _PW_LO and all(
            sum(sizes[:i]) % 8 == 0 for i in range(len(sizes)))
        starts = [sum(sizes[:i]) for i in range(len(sizes))]
        chunks = list(zip(starts, sizes)) + [(_PW_LO, _TAIL)]
        nch = len(chunks)

        def gather(c, slot):
            start, ln = chunks[c]
            src = pl.multiple_of(base + start, 8)
            return pltpu.async_copy(
                x_hbm.at[g_v.at[pl.ds(src, ln)]],
                buf_v.at[slot, pl.ds(0, ln)], semx)

        def out_async(c, slot):
            start, ln = chunks[c]
            return pltpu.async_copy(
                buf_v.at[slot, pl.ds(0, ln)],
                outx_hbm.at[pl.ds(base + start, ln)], semo[slot])

        cps = [gather(0, 0), None]
        outcp = [None] * _NBUF
        look = _NBUF - 1
        pltpu.sync_copy(roif_hbm.at[pl.ds(wid * _BW * _SEG, _BW * _SEG)],
                        in_v)

        lanes = jnp.arange(_L, dtype=jnp.int32)
        b0 = wid * _BW

        def roi_group(bb, grp):
            b = b0 + bb
            tab = plsc.load_gather(g_v, [lanes * _B + (grp * _L * _B + b)])
            s = lax.shift_right_logical(tab, 6)
            for c in range(_DR):
                val = plsc.load_gather(in_v, [s + (bb * _SEG + c * _S)])
                off = bb * _DR * _KO + c * _KO + grp * _L
                if grp * _L + _L <= _KO:
                    out_v[pl.ds(off, _L)] = val
                else:
                    tgt = jnp.minimum(lanes + off, _BW * _DR * _KO - 1)
                    plsc.store_scatter(out_v, [tgt], val,
                                       mask=lanes < _KO - grp * _L)

        roi_tasks = [(bb, grp)
                     for bb in range(_BW) for grp in range(_KPAD // _L)]
        per_iter = -(-len(roi_tasks) // nch)

        for c in range(nch):
            s = c % _NBUF
            if c + look < nch:
                t = (c + look) % _NBUF
                if outcp[t] is not None:
                    outcp[t].wait()
                    outcp[t] = None
                cps[t] = gather(c + look, t)
            for task in roi_tasks[c * per_iter:(c + 1) * per_iter]:
                roi_group(*task)
            cps[s].wait()
            if c < nch - 1:
                outcp[s] = out_async(c, s)
            else:
                @pl.when(wid >= _NW_LO)
                def _():
                    pltpu.sync_copy(
                        buf_v.at[s, pl.ds(0, _TAIL)],
                        outx_hbm.at[pl.ds(base + _PW_LO, _TAIL)])
        for s in range(_NBUF):
            if outcp[s] is not None:
                outcp[s].wait()

        pltpu.sync_copy(out_v,
                        outroi_hbm.at[pl.ds(wid * _BW * _DR * _KO,
                                            _BW * _DR * _KO)])

    return k(xf, roif, g)


def kernel(x, roi):
    xf = x.reshape(_S * _B, _D)
    roif = jnp.transpose(roi, (1, 2, 0)).reshape(-1)
    g = jnp.asarray(_KEEP_ROWS)
    outx, outroi = _sc_gather(xf, roif, g)
    return (outx.reshape(_KO, _B, _D),
            outroi.reshape(_B, _DR, _KO).transpose(2, 0, 1))


# speedup vs baseline: 1.0109x; 1.0109x over previous
import functools

import numpy as np
import jax
import jax.numpy as jnp
from jax import lax
from jax.experimental import pallas as pl
from jax.experimental.pallas import tpu as pltpu
from jax.experimental.pallas import tpu_sc as plsc

_S, _B, _D, _DR = 577, 64, 768, 4
_DRP = 16
_NT = _S - 1
_K = _NT // 2
_KO = _K + 1
_ROWS = _KO * _B
_NC, _NS = 2, 16
_NW = _NC * _NS
_PW_LO, _PW_HI = 576, 584
_NW_LO = 24
_CHUNK = 64
_NBUF = 2
_TAIL = _PW_HI - _PW_LO
_L = 16
_KPAD = 304
_BW = _B // _NW
_SEG = _DR * _S

def _keep_row_table() -> np.ndarray:
    def threefry2x32(k1, k2, x0, x1):
        def rotl(x, r):
            return ((x << np.uint32(r)) | (x >> np.uint32(32 - r))).astype(np.uint32)
        ks = [np.uint32(k1), np.uint32(k2), np.uint32(k1 ^ k2 ^ 0x1BD11BDA)]
        rotations = ((13, 15, 26, 6), (17, 29, 16, 24))
        x0 = (x0 + ks[0]).astype(np.uint32)
        x1 = (x1 + ks[1]).astype(np.uint32)
        for i in range(5):
            for r in rotations[i % 2]:
                x0 = (x0 + x1).astype(np.uint32)
                x1 = rotl(x1, r)
                x1 = (x1 ^ x0).astype(np.uint32)
            x0 = (x0 + ks[(i + 1) % 3]).astype(np.uint32)
            x1 = (x1 + ks[(i + 2) % 3] + np.uint32(i + 1)).astype(np.uint32)
        return x0, x1

    idx = np.arange(_B * _NT, dtype=np.uint64)
    o0, o1 = threefry2x32(0, 1, (idx >> np.uint64(32)).astype(np.uint32),
                          (idx & np.uint64(0xFFFFFFFF)).astype(np.uint32))
    keys = ((o0 ^ o1) >> np.uint32(9)).astype(np.int64).reshape(_B, _NT)
    keep = np.argsort(-keys, axis=1, kind="stable")[:, :_K]
    cols = np.arange(_B, dtype=np.int32)[None, :]
    kept = keep.T.astype(np.int32) * _B + cols
    cls = np.broadcast_to((_S - 1) * _B + cols, (_KPAD - _K, _B))
    return np.concatenate([kept, cls], axis=0).reshape(-1).astype(np.int32)


_KEEP_ROWS = _keep_row_table()


def _worker_base():
    wid = lax.axis_index("s") * _NC + lax.axis_index("c")
    base = wid * _PW_LO + jnp.maximum(wid - _NW_LO, 0) * _TAIL
    return wid, base


def _sc_gather(xf, roif, g):
    mesh = plsc.VectorSubcoreMesh(core_axis_name="c", subcore_axis_name="s")
    nfull = _PW_LO // _CHUNK

    @functools.partial(
        pl.kernel,
        mesh=mesh,
        out_type=(
            jax.ShapeDtypeStruct((_ROWS, _D), jnp.float32),
            jax.ShapeDtypeStruct((_B * _DR * _KO,), jnp.float32),
        ),
        scratch_types=[
            pltpu.VMEM((_KPAD * _B,), jnp.int32),
            pltpu.VMEM((_NBUF, _CHUNK, _D), jnp.float32),
            pltpu.VMEM((_BW * _SEG,), jnp.float32),
            pltpu.VMEM((_BW * _DR * _KO,), jnp.float32),
            pltpu.SemaphoreType.DMA,
            [pltpu.SemaphoreType.DMA] * _NBUF,
        ],
        compiler_params=pltpu.CompilerParams(needs_layout_passes=False),
    )
    def k(x_hbm, roif_hbm, g_hbm, outx_hbm, outroi_hbm,
          g_v, buf_v, in_v, out_v, semx, semo):
        wid, base = _worker_base()
        pltpu.sync_copy(g_hbm, g_v)

        chunks = [(i * _CHUNK, _CHUNK) for i in range(nfull)] + [(_PW_LO, _TAIL)]
        assert all(s % 8 == 0 for s, _ in chunks) and nfull * _CHUNK == _PW_LO
        nch = len(chunks)

        def gather(c, slot):
            start, ln = chunks[c]
            src = pl.multiple_of(base + start, 8)
            return pltpu.async_copy(
                x_hbm.at[g_v.at[pl.ds(src, ln)]],
                buf_v.at[slot, pl.ds(0, ln)], semx)

        def out_async(c, slot):
            start, ln = chunks[c]
            return pltpu.async_copy(
                buf_v.at[slot, pl.ds(0, ln)],
                outx_hbm.at[pl.ds(base + start, ln)], semo[slot])

        cps = [gather(0, 0), None]
        outcp = [None] * _NBUF
        look = _NBUF - 1
        pltpu.sync_copy(roif_hbm.at[pl.ds(wid * _BW * _SEG, _BW * _SEG)],
                        in_v)

        lanes = jnp.arange(_L, dtype=jnp.int32)
        b0 = wid * _BW

        def roi_group(bb, grp):
            b = b0 + bb
            tab = plsc.load_gather(g_v, [lanes * _B + (grp * _L * _B + b)])
            s = lax.shift_right_logical(tab, 6)
            for c in range(_DR):
                val = plsc.load_gather(in_v, [s + (bb * _SEG + c * _S)])
                off = bb * _DR * _KO + c * _KO + grp * _L
                if grp * _L + _L <= _KO:
                    out_v[pl.ds(off, _L)] = val
                else:
                    tgt = jnp.minimum(lanes + off, _BW * _DR * _KO - 1)
                    plsc.store_scatter(out_v, [tgt], val,
                                       mask=lanes < _KO - grp * _L)

        roi_tasks = [(bb, grp)
                     for bb in range(_BW) for grp in range(_KPAD // _L)]
        per_iter = -(-len(roi_tasks) // nch)

        for c in range(nch):
            s = c % _NBUF
            if c + look < nch:
                t = (c + look) % _NBUF
                if outcp[t] is not None:
                    outcp[t].wait()
                    outcp[t] = None
                cps[t] = gather(c + look, t)
            for task in roi_tasks[c * per_iter:(c + 1) * per_iter]:
                roi_group(*task)
            cps[s].wait()
            if c < nch - 1:
                outcp[s] = out_async(c, s)
            else:
                @pl.when(wid >= _NW_LO)
                def _():
                    pltpu.sync_copy(
                        buf_v.at[s, pl.ds(0, _TAIL)],
                        outx_hbm.at[pl.ds(base + _PW_LO, _TAIL)])
        for s in range(_NBUF):
            if outcp[s] is not None:
                outcp[s].wait()

        pltpu.sync_copy(out_v,
                        outroi_hbm.at[pl.ds(wid * _BW * _DR * _KO,
                                            _BW * _DR * _KO)])

    return k(xf, roif, g)


def kernel(x, roi):
    xf = x.reshape(_S * _B, _D)
    roif = jnp.transpose(roi, (1, 2, 0)).reshape(-1)
    g = jnp.asarray(_KEEP_ROWS)
    outx, outroi = _sc_gather(xf, roif, g)
    return (outx.reshape(_KO, _B, _D),
            outroi.reshape(_B, _DR, _KO).transpose(2, 0, 1))
